# baseline (speedup 1.0000x reference)
import numpy as np
import jax
import jax.numpy as jnp
from jax.experimental import pallas as pl
from jax.experimental.pallas import tpu as pltpu

_FD = 5
_NS = 50
_KC = 2 * _FD + 1


def _fourier_block_matrix():
    k = np.arange(-_FD, _FD + 1, dtype=np.float32).reshape(-1, 1)
    i = np.arange(0, _NS, dtype=np.float32).reshape(1, -1)
    T = np.float32(2.0 * np.pi / _NS) * (k @ i)
    cosT = np.cos(T).astype(np.float32)
    sinT = np.sin(T).astype(np.float32)
    M = np.zeros((2 * _NS, 2 * _KC), dtype=np.float32)
    M[:_NS, :_KC] = cosT.T
    M[:_NS, _KC:] = -sinT.T
    M[_NS:, :_KC] = sinT.T
    M[_NS:, _KC:] = cosT.T
    return M


_M_CONST = _fourier_block_matrix()


def _selector_matrix():
    S = np.zeros((8, 2 * _NS), dtype=np.float32)
    S[0, :_NS] = 1.0
    S[1, _NS:] = 1.0
    return S


_S_CONST = _selector_matrix()


def _ce2(l0, l1, lab):
    m = jnp.maximum(l0, l1)
    lse = m + jnp.log(jnp.exp(l0 - m) + jnp.exp(l1 - m))
    picked = jnp.where(lab > 0.5, l1, l0)
    return lse - picked


def _level_body(mmat, smat, cls_ref, reg_ref, mask_ref, map_ref):
    cls = cls_ref[0]
    msk = mask_ref[0].astype(jnp.float32)
    tr_mask = msk[0:1]
    tcl_mask = msk[1:2]
    train = msk[2:3]

    ce_tr = _ce2(cls[0:1], cls[1:2], tr_mask)
    ce_tcl = _ce2(cls[2:3], cls[3:4], tcl_mask)
    posf = ((tr_mask * train) > 0.5).astype(jnp.float32)
    negb = ((1.0 - tr_mask) * train) > 0.5
    negce = jnp.where(negb, ce_tr, -1e30)

    wv = posf * (tr_mask + tcl_mask) * 0.5

    Xd = reg_ref[0] - map_ref[0]
    D = jnp.dot(mmat, Xd, preferred_element_type=jnp.float32)
    ad = jnp.abs(D)
    mn = jnp.minimum(ad, 1.0)
    sl1 = mn * (ad - 0.5 * mn)
    R = jnp.dot(smat, sl1, preferred_element_type=jnp.float32)

    part = jnp.concatenate([
        posf,
        posf * ce_tr,
        negb.astype(jnp.float32),
        ce_tcl,
        ce_tcl * posf,
        R[0:1] * wv,
        R[1:2] * wv,
        jnp.zeros_like(posf),
    ], axis=0)
    return jnp.sum(part, axis=1, keepdims=True), negce


def _interleaved_topk_sums(nc3, nc4, nc5, k3, k4, k5):
    def keys_of(nc):
        kb = jax.lax.bitcast_convert_type(nc, jnp.int32)
        return jnp.where(nc >= 0.0, kb, jnp.int32(-1))

    q3, q4, q5 = keys_of(nc3), keys_of(nc4), keys_of(nc5)

    def refine(q, k, t, p):
        c1 = jnp.sum((q >= (t | (jnp.int32(1) << p))).astype(jnp.int32))
        c2 = jnp.sum((q >= (t | (jnp.int32(2) << p))).astype(jnp.int32))
        c3_ = jnp.sum((q >= (t | (jnp.int32(3) << p))).astype(jnp.int32))
        d = ((c1 >= k).astype(jnp.int32) + (c2 >= k).astype(jnp.int32)
             + (c3_ >= k).astype(jnp.int32))
        return t | (d << p)

    def step(i, ts):
        t3, t4, t5 = ts
        p = 28 - 2 * i
        return (refine(q3, k3, t3, p), refine(q4, k4, t4, p),
                refine(q5, k5, t5, p))

    z = jnp.int32(0)
    def top(q, k):
        b = jnp.int32(1) << 30
        c = jnp.sum((q >= b).astype(jnp.int32))
        return jnp.where(c >= k, b, z)

    tf3, tf4, tf5 = jax.lax.fori_loop(
        0, 15, step, (top(q3, k3), top(q4, k4), top(q5, k5)))

    def finish(nc, q, tf, k):
        thr = jax.lax.bitcast_convert_type(tf, jnp.float32)
        gt = q > tf
        cnt_gt = jnp.sum(gt.astype(jnp.float32))
        sum_gt = jnp.sum(jnp.where(gt, nc, 0.0))
        loss_neg = sum_gt + (k.astype(jnp.float32) - cnt_gt) * thr
        return jnp.where(k > 0, loss_neg, 0.0)

    return finish(nc3, q3, tf3, k3), finish(nc4, q4, tf4, k4), \
        finish(nc5, q5, tf5, k5)


def _level_losses(srow, loss_neg, n_negf, npix):
    n_pos = srow[0, 0]
    sum_pos_ce = srow[1, 0]
    stcl_neg = srow[3, 0] - srow[4, 0]
    stcl_pos = srow[4, 0]
    sm_neg = np.float32(npix) - n_pos
    rx = srow[5, 0]
    ry = srow[6, 0]

    loss_tr = (sum_pos_ce + loss_neg) / jnp.maximum(n_pos + n_negf, 1.0)
    loss_tcl = jnp.where(
        n_pos > 0,
        stcl_pos / jnp.maximum(n_pos, 1.0)
        + 0.5 * stcl_neg / jnp.maximum(sm_neg, 1.0),
        0.0)
    denom = jnp.maximum(n_pos * np.float32(_NS), 1.0)
    lrx = jnp.where(n_pos > 0, rx / denom, 0.0)
    lry = jnp.where(n_pos > 0, ry / denom, 0.0)
    return loss_tr, loss_tcl, lrx, lry


def _n_neg(srow):
    n_pos = srow[0, 0]
    num_neg = srow[2, 0].astype(jnp.int32)
    return jnp.where(n_pos > 0,
                     jnp.minimum(num_neg, (3.0 * n_pos).astype(jnp.int32)),
                     jnp.minimum(num_neg, 100))


def _fused_kernel(mmat_ref, smat_ref,
                  cls3_ref, reg3_ref, msk3_ref, map3_ref,
                  cls4_ref, reg4_ref, msk4_ref, map4_ref,
                  cls5_ref, reg5_ref, msk5_ref, map5_ref,
                  out_ref,
                  nc3_ref, nc4_ref, nc5_ref, a3_ref, a4_ref, a5_ref):
    b = pl.program_id(0)
    t = pl.program_id(1)
    nb = pl.num_programs(0)
    nt = pl.num_programs(1)
    mmat = mmat_ref[...]
    smat = smat_ref[...]

    @pl.when((b == 0) & (t == 0))
    def _():
        a3_ref[...] = jnp.zeros_like(a3_ref)
        a4_ref[...] = jnp.zeros_like(a4_ref)
        a5_ref[...] = jnp.zeros_like(a5_ref)

    s3, n3 = _level_body(mmat, smat, cls3_ref, reg3_ref, msk3_ref, map3_ref)
    s4, n4 = _level_body(mmat, smat, cls4_ref, reg4_ref, msk4_ref, map4_ref)
    s5, n5 = _level_body(mmat, smat, cls5_ref, reg5_ref, msk5_ref, map5_ref)
    h3 = n3.shape[1]
    h4 = n4.shape[1]
    h5 = n5.shape[1]
    nc3_ref[pl.ds(b, 1), pl.ds(t * h3, h3)] = n3
    nc4_ref[pl.ds(b, 1), pl.ds(t * h4, h4)] = n4
    nc5_ref[pl.ds(b, 1), pl.ds(t * h5, h5)] = n5
    a3_ref[:, 0:1] += s3
    a4_ref[:, 0:1] += s4
    a5_ref[:, 0:1] += s5

    @pl.when((b == nb - 1) & (t == nt - 1))
    def _():
        r3 = a3_ref[:, 0:1]
        r4 = a4_ref[:, 0:1]
        r5 = a5_ref[:, 0:1]
        k3, k4, k5 = _n_neg(r3), _n_neg(r4), _n_neg(r5)
        ln3, ln4, ln5 = _interleaved_topk_sums(
            nc3_ref[...], nc4_ref[...], nc5_ref[...], k3, k4, k5)
        l3 = _level_losses(r3, ln3, k3.astype(jnp.float32), 8 * 128 * 128)
        l4 = _level_losses(r4, ln4, k4.astype(jnp.float32), 8 * 64 * 64)
        l5 = _level_losses(r5, ln5, k5.astype(jnp.float32), 8 * 32 * 32)
        loss_tr = l3[0] + l4[0] + l5[0]
        loss_tcl = l3[1] + l4[1] + l5[1]
        loss_rx = l3[2] + l4[2] + l5[2]
        loss_ry = l3[3] + l4[3] + l5[3]
        loss_all = loss_tr + loss_tcl + loss_rx + loss_ry
        zero = jnp.float32(0.0)
        out_ref[...] = jnp.stack(
            [loss_all, loss_tr, loss_tcl, loss_rx, loss_ry,
             zero, zero, zero]).reshape(1, 8)


def kernel(cls_p3, reg_p3, mask_p3, map_p3, cls_p4, reg_p4, mask_p4, map_p4,
           cls_p5, reg_p5, mask_p5, map_p5):
    mmat = jnp.asarray(_M_CONST)
    smat = jnp.asarray(_S_CONST)
    B = cls_p3.shape[0]
    HW3 = cls_p3.shape[2] * cls_p3.shape[3]
    HW4 = cls_p4.shape[2] * cls_p4.shape[3]
    HW5 = cls_p5.shape[2] * cls_p5.shape[3]

    NT = 2

    def lvl_specs(hw):
        p = hw // NT
        return [
            pl.BlockSpec((1, 4, p), lambda b, t: (b, 0, t)),
            pl.BlockSpec((1, 22, p), lambda b, t: (b, 0, t)),
            pl.BlockSpec((1, 3, p), lambda b, t: (b, 0, t)),
            pl.BlockSpec((1, 22, p), lambda b, t: (b, 0, t)),
        ]

    out = pl.pallas_call(
        _fused_kernel,
        grid=(B, NT),
        in_specs=[
            pl.BlockSpec((2 * _NS, 2 * _KC), lambda b, t: (0, 0)),
            pl.BlockSpec((8, 2 * _NS), lambda b, t: (0, 0)),
        ] + lvl_specs(HW3) + lvl_specs(HW4) + lvl_specs(HW5),
        out_specs=pl.BlockSpec((1, 8), lambda b, t: (0, 0)),
        out_shape=jax.ShapeDtypeStruct((1, 8), jnp.float32),
        scratch_shapes=[
            pltpu.VMEM((B, HW3), jnp.float32),
            pltpu.VMEM((B, HW4), jnp.float32),
            pltpu.VMEM((B, HW5), jnp.float32),
            pltpu.VMEM((8, 128), jnp.float32),
            pltpu.VMEM((8, 128), jnp.float32),
            pltpu.VMEM((8, 128), jnp.float32),
        ],
    )(mmat, smat,
      cls_p3.reshape(B, 4, HW3), reg_p3.reshape(B, 22, HW3),
      mask_p3.reshape(B, 3, HW3), map_p3.reshape(B, 22, HW3),
      cls_p4.reshape(B, 4, HW4), reg_p4.reshape(B, 22, HW4),
      mask_p4.reshape(B, 3, HW4), map_p4.reshape(B, 22, HW4),
      cls_p5.reshape(B, 4, HW5), reg_p5.reshape(B, 22, HW5),
      mask_p5.reshape(B, 3, HW5), map_p5.reshape(B, 22, HW5))
    return out[0, :5]


# speedup vs baseline: 1.0107x; 1.0107x over previous
import numpy as np
import jax
import jax.numpy as jnp
from jax.experimental import pallas as pl
from jax.experimental.pallas import tpu as pltpu

_FD = 5
_NS = 50
_KC = 2 * _FD + 1


def _fourier_block_matrix():
    k = np.arange(-_FD, _FD + 1, dtype=np.float32).reshape(-1, 1)
    i = np.arange(0, _NS, dtype=np.float32).reshape(1, -1)
    T = np.float32(2.0 * np.pi / _NS) * (k @ i)
    cosT = np.cos(T).astype(np.float32)
    sinT = np.sin(T).astype(np.float32)
    M = np.zeros((2 * _NS, 2 * _KC), dtype=np.float32)
    M[:_NS, :_KC] = cosT.T
    M[:_NS, _KC:] = -sinT.T
    M[_NS:, :_KC] = sinT.T
    M[_NS:, _KC:] = cosT.T
    return M


_M_CONST = _fourier_block_matrix()


def _selector_matrix():
    S = np.zeros((8, 2 * _NS), dtype=np.float32)
    S[0, :_NS] = 1.0
    S[1, _NS:] = 1.0
    return S


_S_CONST = _selector_matrix()


def _ce2(l0, l1, lab):
    m = jnp.maximum(l0, l1)
    lse = m + jnp.log(jnp.exp(l0 - m) + jnp.exp(l1 - m))
    picked = jnp.where(lab > 0.5, l1, l0)
    return lse - picked


def _level_body(mmat, smat, cls_ref, reg_ref, mask_ref, map_ref):
    cls = cls_ref[0]
    msk = mask_ref[0].astype(jnp.float32)
    tr_mask = msk[0:1]
    tcl_mask = msk[1:2]
    train = msk[2:3]

    ce_tr = _ce2(cls[0:1], cls[1:2], tr_mask)
    ce_tcl = _ce2(cls[2:3], cls[3:4], tcl_mask)
    posf = ((tr_mask * train) > 0.5).astype(jnp.float32)
    negb = ((1.0 - tr_mask) * train) > 0.5
    negce = jnp.where(negb, ce_tr, -1e30)

    wv = posf * (tr_mask + tcl_mask) * 0.5

    Xd = reg_ref[0] - map_ref[0]
    D = jnp.dot(mmat, Xd, preferred_element_type=jnp.float32)
    ad = jnp.abs(D)
    mn = jnp.minimum(ad, 1.0)
    sl1 = mn * (ad - 0.5 * mn)
    R = jnp.dot(smat, sl1, preferred_element_type=jnp.float32)

    part = jnp.concatenate([
        posf,
        posf * ce_tr,
        negb.astype(jnp.float32),
        ce_tcl,
        ce_tcl * posf,
        R[0:1] * wv,
        R[1:2] * wv,
        jnp.zeros_like(posf),
    ], axis=0)
    return jnp.sum(part, axis=1, keepdims=True), negce


def _interleaved_topk_sums(nc3, nc4, nc5, k3, k4, k5):
    def keys_of(nc):
        kb = jax.lax.bitcast_convert_type(nc, jnp.int32)
        return jnp.where(nc >= 0.0, kb, jnp.int32(-1))

    q3, q4, q5 = keys_of(nc3), keys_of(nc4), keys_of(nc5)

    def refine(q, k, t, p):
        c1 = jnp.sum((q >= (t | (jnp.int32(1) << p))).astype(jnp.int32))
        c2 = jnp.sum((q >= (t | (jnp.int32(2) << p))).astype(jnp.int32))
        c3_ = jnp.sum((q >= (t | (jnp.int32(3) << p))).astype(jnp.int32))
        d = ((c1 >= k).astype(jnp.int32) + (c2 >= k).astype(jnp.int32)
             + (c3_ >= k).astype(jnp.int32))
        return t | (d << p)

    def step(i, ts):
        t3, t4, t5 = ts
        p = 28 - 2 * i
        return (refine(q3, k3, t3, p), refine(q4, k4, t4, p),
                refine(q5, k5, t5, p))

    z = jnp.int32(0)
    def top(q, k):
        b = jnp.int32(1) << 30
        c = jnp.sum((q >= b).astype(jnp.int32))
        return jnp.where(c >= k, b, z)

    tf3, tf4, tf5 = jax.lax.fori_loop(
        0, 15, step, (top(q3, k3), top(q4, k4), top(q5, k5)))

    def finish(nc, q, tf, k):
        thr = jax.lax.bitcast_convert_type(tf, jnp.float32)
        gt = q > tf
        cnt_gt = jnp.sum(gt.astype(jnp.float32))
        sum_gt = jnp.sum(jnp.where(gt, nc, 0.0))
        loss_neg = sum_gt + (k.astype(jnp.float32) - cnt_gt) * thr
        return jnp.where(k > 0, loss_neg, 0.0)

    return finish(nc3, q3, tf3, k3), finish(nc4, q4, tf4, k4), \
        finish(nc5, q5, tf5, k5)


def _level_losses(srow, loss_neg, n_negf, npix):
    n_pos = srow[0, 0]
    sum_pos_ce = srow[1, 0]
    stcl_neg = srow[3, 0] - srow[4, 0]
    stcl_pos = srow[4, 0]
    sm_neg = np.float32(npix) - n_pos
    rx = srow[5, 0]
    ry = srow[6, 0]

    loss_tr = (sum_pos_ce + loss_neg) / jnp.maximum(n_pos + n_negf, 1.0)
    loss_tcl = jnp.where(
        n_pos > 0,
        stcl_pos / jnp.maximum(n_pos, 1.0)
        + 0.5 * stcl_neg / jnp.maximum(sm_neg, 1.0),
        0.0)
    denom = jnp.maximum(n_pos * np.float32(_NS), 1.0)
    lrx = jnp.where(n_pos > 0, rx / denom, 0.0)
    lry = jnp.where(n_pos > 0, ry / denom, 0.0)
    return loss_tr, loss_tcl, lrx, lry


def _n_neg(srow):
    n_pos = srow[0, 0]
    num_neg = srow[2, 0].astype(jnp.int32)
    return jnp.where(n_pos > 0,
                     jnp.minimum(num_neg, (3.0 * n_pos).astype(jnp.int32)),
                     jnp.minimum(num_neg, 100))


def _fused_kernel(mmat_ref, smat_ref,
                  cls3_ref, reg3_ref, msk3_ref, map3_ref,
                  cls4_ref, reg4_ref, msk4_ref, map4_ref,
                  cls5_ref, reg5_ref, msk5_ref, map5_ref,
                  out_ref,
                  nc3_ref, nc4_ref, nc5_ref, a3_ref, a4_ref, a5_ref):
    b = pl.program_id(0)
    nb = pl.num_programs(0)
    mmat = mmat_ref[...]
    smat = smat_ref[...]

    @pl.when(b == 0)
    def _():
        a3_ref[...] = jnp.zeros_like(a3_ref)
        a4_ref[...] = jnp.zeros_like(a4_ref)
        a5_ref[...] = jnp.zeros_like(a5_ref)

    s3, n3 = _level_body(mmat, smat, cls3_ref, reg3_ref, msk3_ref, map3_ref)
    s4, n4 = _level_body(mmat, smat, cls4_ref, reg4_ref, msk4_ref, map4_ref)
    s5, n5 = _level_body(mmat, smat, cls5_ref, reg5_ref, msk5_ref, map5_ref)
    nc3_ref[pl.ds(b, 1), :] = n3
    nc4_ref[pl.ds(b, 1), :] = n4
    nc5_ref[pl.ds(b, 1), :] = n5
    a3_ref[:, 0:1] += s3
    a4_ref[:, 0:1] += s4
    a5_ref[:, 0:1] += s5

    @pl.when(b == nb - 1)
    def _():
        r3 = a3_ref[:, 0:1]
        r4 = a4_ref[:, 0:1]
        r5 = a5_ref[:, 0:1]
        k3, k4, k5 = _n_neg(r3), _n_neg(r4), _n_neg(r5)
        ln3, ln4, ln5 = _interleaved_topk_sums(
            nc3_ref[...], nc4_ref[...], nc5_ref[...], k3, k4, k5)
        l3 = _level_losses(r3, ln3, k3.astype(jnp.float32), 8 * 128 * 128)
        l4 = _level_losses(r4, ln4, k4.astype(jnp.float32), 8 * 64 * 64)
        l5 = _level_losses(r5, ln5, k5.astype(jnp.float32), 8 * 32 * 32)
        loss_tr = l3[0] + l4[0] + l5[0]
        loss_tcl = l3[1] + l4[1] + l5[1]
        loss_rx = l3[2] + l4[2] + l5[2]
        loss_ry = l3[3] + l4[3] + l5[3]
        loss_all = loss_tr + loss_tcl + loss_rx + loss_ry
        zero = jnp.float32(0.0)
        out_ref[...] = jnp.stack(
            [loss_all, loss_tr, loss_tcl, loss_rx, loss_ry,
             zero, zero, zero]).reshape(1, 8)


def kernel(cls_p3, reg_p3, mask_p3, map_p3, cls_p4, reg_p4, mask_p4, map_p4,
           cls_p5, reg_p5, mask_p5, map_p5):
    mmat = jnp.asarray(_M_CONST)
    smat = jnp.asarray(_S_CONST)
    B = cls_p3.shape[0]
    HW3 = cls_p3.shape[2] * cls_p3.shape[3]
    HW4 = cls_p4.shape[2] * cls_p4.shape[3]
    HW5 = cls_p5.shape[2] * cls_p5.shape[3]

    def lvl_specs(hw):
        return [
            pl.BlockSpec((1, 4, hw), lambda b: (b, 0, 0)),
            pl.BlockSpec((1, 22, hw), lambda b: (b, 0, 0)),
            pl.BlockSpec((1, 3, hw), lambda b: (b, 0, 0)),
            pl.BlockSpec((1, 22, hw), lambda b: (b, 0, 0)),
        ]

    out = pl.pallas_call(
        _fused_kernel,
        grid=(B,),
        in_specs=[
            pl.BlockSpec((2 * _NS, 2 * _KC), lambda b: (0, 0)),
            pl.BlockSpec((8, 2 * _NS), lambda b: (0, 0)),
        ] + lvl_specs(HW3) + lvl_specs(HW4) + lvl_specs(HW5),
        out_specs=pl.BlockSpec((1, 8), lambda b: (0, 0)),
        out_shape=jax.ShapeDtypeStruct((1, 8), jnp.float32),
        scratch_shapes=[
            pltpu.VMEM((B, HW3), jnp.float32),
            pltpu.VMEM((B, HW4), jnp.float32),
            pltpu.VMEM((B, HW5), jnp.float32),
            pltpu.VMEM((8, 128), jnp.float32),
            pltpu.VMEM((8, 128), jnp.float32),
            pltpu.VMEM((8, 128), jnp.float32),
        ],
    )(mmat, smat,
      cls_p3.reshape(B, 4, HW3), reg_p3.reshape(B, 22, HW3),
      mask_p3.reshape(B, 3, HW3), map_p3.reshape(B, 22, HW3),
      cls_p4.reshape(B, 4, HW4), reg_p4.reshape(B, 22, HW4),
      mask_p4.reshape(B, 3, HW4), map_p4.reshape(B, 22, HW4),
      cls_p5.reshape(B, 4, HW5), reg_p5.reshape(B, 22, HW5),
      mask_p5.reshape(B, 3, HW5), map_p5.reshape(B, 22, HW5))
    return out[0, :5]
